# baseline (speedup 1.0000x reference)
import functools

import jax
import jax.numpy as jnp
from jax import lax
from jax.experimental import pallas as pl
from jax.experimental.pallas import tpu as pltpu
from jax.experimental.pallas import tpu_sc as plsc

B = 16384
D = 64
NE = 1000000
NC = 2
NS = 16
NW = NC * NS
BPW = B // NW
C2 = 32
NCHUNK2 = BPW // C2
NV = D // 16

NBKT = 7812
ETAIL = NBKT * 128
BPT = 248
WB = 4
NWAVE = BPT // WB
EPT = BPT * 128
LCAP = 1024
SCAP = 48
BOUT = B + SCAP


def _iota16():
  return lax.iota(jnp.int32, 16)


def _splat(v):
  return jnp.full((16,), v, jnp.int32)


def _sc_gather(a1, r1, a2, r2, ent3, relp, offp, tailp):
  mesh = plsc.VectorSubcoreMesh(core_axis_name="c", subcore_axis_name="s")
  out_small = jax.ShapeDtypeStruct((B, 2 * D), jnp.float32)
  out_big = jax.ShapeDtypeStruct((BOUT, 2 * D), jnp.float32)

  @functools.partial(
      pl.kernel,
      out_type=[out_small, out_small, out_big, out_big],
      mesh=mesh,
      compiler_params=pltpu.CompilerParams(needs_layout_passes=False),
      scratch_types=[
          pltpu.VMEM((BPW,), jnp.int32),
          pltpu.VMEM((BPW,), jnp.int32),
          pltpu.VMEM((BPW,), jnp.int32),
          pltpu.VMEM((BPW,), jnp.int32),
          pltpu.VMEM((C2, 2 * D), jnp.float32),
          pltpu.VMEM((C2, 2 * D), jnp.float32),
          pltpu.VMEM((C2, 2 * D), jnp.float32),
          pltpu.VMEM((C2, 2 * D), jnp.float32),
          pltpu.VMEM((C2, 2 * D), jnp.float32),
          pltpu.VMEM((C2, 2 * D), jnp.float32),
          pltpu.VMEM((16,), jnp.int32),
          pltpu.VMEM((16,), jnp.int32),
          pltpu.VMEM((16, 2 * D), jnp.float32),
          pltpu.VMEM((4096,), jnp.int32),
          pltpu.VMEM((LCAP,), jnp.int32),
          pltpu.VMEM((LCAP,), jnp.int32),
          pltpu.VMEM((LCAP,), jnp.int32),
          pltpu.VMEM((LCAP,), jnp.int32),
          pltpu.VMEM((SCAP,), jnp.int32),
          pltpu.VMEM((SCAP,), jnp.int32),
          pltpu.VMEM((SCAP,), jnp.int32),
          pltpu.VMEM((SCAP, 2 * D), jnp.float32),
          pltpu.VMEM((SCAP, 2 * D), jnp.float32),
          pltpu.VMEM((8, 8, 2 * WB * 128), jnp.float32),
          pltpu.SemaphoreType.DMA,
          pltpu.SemaphoreType.DMA,
      ],
  )
  def k(a1_h, r1_h, a2_h, r2_h, ent_h, rel_h, off_h, tail_h,
        ro_o, po_o, eo1_o, eo2_o,
        ia1, ia2, ir1, ir2, rb1, rb2, ob1, ob2, re, pp,
        te, tp, tb, av, le1, lp1, le2, lp2, se, sp1, sp2, stg1, stg2,
        ring, sem, sem2):
    wid = lax.axis_index("s") * NC + lax.axis_index("c")
    base_w = wid * BPW
    i16 = _iota16()

    pltpu.sync_copy(a1_h.at[pl.ds(base_w, BPW)], ia1)
    pltpu.sync_copy(a2_h.at[pl.ds(base_w, BPW)], ia2)
    pltpu.sync_copy(r1_h.at[pl.ds(base_w, BPW)], ir1)
    pltpu.sync_copy(r2_h.at[pl.ds(base_w, BPW)], ir2)
    for ci in range(NCHUNK2):
      o = ci * C2
      gcps = [
          pltpu.async_copy(rel_h.at[ir1.at[pl.ds(o, C2)]], rb1, sem),
          pltpu.async_copy(rel_h.at[ir2.at[pl.ds(o, C2)]], rb2, sem),
          pltpu.async_copy(off_h.at[ir1.at[pl.ds(o, C2)]], ob1, sem),
          pltpu.async_copy(off_h.at[ir2.at[pl.ds(o, C2)]], ob2, sem),
      ]
      for cp in gcps:
        cp.wait()

      def row_body(rr, _):
        for j in range(NV):
          sl = pl.ds(j * 16, 16)
          sr = pl.ds(D + j * 16, 16)
          re[rr, sl] = rb1[rr, sl]
          re[rr, sr] = rb2[rr, sl]
          pp[rr, sl] = ob1[rr, sl]
          pp[rr, sr] = ob2[rr, sl]
        return 0

      lax.fori_loop(0, C2, row_body, 0, unroll=2)
      base = base_w + o
      pltpu.sync_copy(re, ro_o.at[pl.ds(base, C2)])
      pltpu.sync_copy(pp, po_o.at[pl.ds(base, C2)])

    for ia, eo_h in ((ia1, eo1_o), (ia2, eo2_o)):
      for si in range(1):
        te[pl.ds(0, 16)] = _splat(0)
        tp[pl.ds(0, 16)] = _splat(B) + i16

      def tail_scan(g, nt):
        ev = ia[pl.ds(g * 16, 16)]
        m = ev >= ETAIL
        c = plsc.cumsum(jnp.where(m, 1, 0))
        idx = nt + c - 1
        m2 = jnp.logical_and(m, idx < 16)
        plsc.store_scatter(te, [idx], ev - ETAIL, mask=m2)
        plsc.store_scatter(tp, [idx], _splat(base_w) + g * 16 + i16, mask=m2)
        return nt + plsc.all_reduce_population_count(m)[0]

      lax.fori_loop(0, BPW // 16, tail_scan, jnp.int32(0))
      pltpu.async_copy(tail_h.at[te], tb, sem).wait()
      pltpu.async_copy(tb, eo_h.at[tp], sem).wait()

    lo = wid * EPT
    hi = lo + EPT
    nql = []
    for le, lp, a_h in ((le1, lp1, a1_h), (le2, lp2, a2_h)):
      def prefill(j, _):
        le[pl.ds(j * 16, 16)] = _splat(2 ** 30)
        return 0

      lax.fori_loop(0, LCAP // 16, prefill, 0)
      n0 = jnp.int32(0)
      for half in range(4):
        pltpu.sync_copy(a_h.at[pl.ds(half * 4096, 4096)], av)

        def scan(g, n):
          ev = av[pl.ds(g * 16, 16)]
          m = jnp.logical_and(jnp.logical_and(ev >= lo, ev < hi), ev < ETAIL)
          c = plsc.cumsum(jnp.where(m, 1, 0))
          idx = n + c - 1
          m2 = jnp.logical_and(m, idx < LCAP)
          plsc.store_scatter(le, [idx], ev, mask=m2)
          pos = _splat(half * 4096) + g * 16 + i16
          plsc.store_scatter(lp, [idx], pos, mask=m2)
          return n + plsc.all_reduce_population_count(m)[0]

        n0 = lax.fori_loop(0, 4096 // 16, scan, n0)
      nql.append(jnp.minimum(n0, LCAP))

    bkt_base = wid * BPT
    ng1 = lax.shift_right_logical(nql[0] + 15, 4)
    ng2 = lax.shift_right_logical(nql[1] + 15, 4)
    HL = WB * 128

    def fetch(g):
      b0 = jnp.minimum(bkt_base + g * WB, NBKT - WB)
      start = pl.multiple_of(b0 * 128, 128)
      hoff = pl.multiple_of(jnp.bitwise_and(g, 1) * HL, 128)
      for a in range(8):
        pltpu.async_copy(ent_h.at[pl.ds(a, 1), :, pl.ds(start, HL)],
                         ring.at[pl.ds(a, 1), :, pl.ds(hoff, HL)], sem)

    fetch(jnp.int32(0))

    def wave(g, _):
      @pl.when(g < NWAVE - 1)
      def _():
        fetch(g + 1)

      for a in range(8):
        pltpu.make_async_copy(ent_h.at[pl.ds(0, 1), :, pl.ds(0, HL)],
                              ring.at[pl.ds(0, 1), :, pl.ds(0, HL)],
                              sem).wait()
      wv_t = wid * NWAVE + g
      coff = jnp.bitwise_and(g, 1) * HL
      for le, lp, eo_h, sp, stg, ng in (
          (le1, lp1, eo1_o, sp1, stg1, ng1),
          (le2, lp2, eo2_o, sp2, stg2, ng2)):
        @pl.when(g > 0)
        def _():
          pltpu.make_async_copy(stg, eo_h.at[sp], sem2).wait()

        for si in range(SCAP // 16):
          se[pl.ds(si * 16, 16)] = _splat(bkt_base * 128)
          sp[pl.ds(si * 16, 16)] = _splat(B + si * 16) + i16

        def stage(j, ns):
          ev = le[pl.ds(j * 16, 16)]
          pv = lp[pl.ds(j * 16, 16)]
          m = lax.shift_right_logical(ev, 9) == wv_t
          c = plsc.cumsum(jnp.where(m, 1, 0))
          idx = ns + c - 1
          m2 = jnp.logical_and(m, idx < SCAP)
          plsc.store_scatter(se, [idx], ev, mask=m2)
          plsc.store_scatter(sp, [idx], pv, mask=m2)
          return ns + plsc.all_reduce_population_count(m)[0]

        ns = lax.fori_loop(0, ng * 0, stage, jnp.int32(0))

        def extract(kk, _):
          ev = se[pl.ds(kk * 16, 16)]
          col = jnp.bitwise_and(lax.shift_right_logical(ev, 7),
                                _splat(WB - 1)) * 128 \
              + jnp.bitwise_and(ev, _splat(127)) + coff
          qv = _splat(0) + kk * 16 + i16
          for d in range(D):
            vals = plsc.load_gather(ring, [_splat(d >> 3), _splat(d & 7), col])
            plsc.store_scatter(stg, [qv, _splat(d)], vals)
          return 0

        nk = lax.shift_right_logical(jnp.minimum(ns, SCAP) + 15, 4)
        lax.fori_loop(0, nk, extract, 0)
        pltpu.async_copy(stg, eo_h.at[sp], sem2)
      return 0

    lax.fori_loop(0, NWAVE, wave, 0)
    for eo_h, sp, stg in ((eo1_o, sp1, stg1), (eo2_o, sp2, stg2)):
      pltpu.make_async_copy(stg, eo_h.at[sp], sem2).wait()

  return k(a1, r1, a2, r2, ent3, relp, offp, tailp)


BK = 2048


def _tc_body(eo1, eo2, ro, po, cw1, cb1, cw2, cb2, ow1, ob1, ow2, ob2,
             center_o, offset_o):
  dn = (((1,), (1,)), ((), ()))
  emb1 = eo1[:, :D] + ro[:, :D]
  emb2 = eo2[:, :D] + ro[:, D:]
  cb1v = cb1[...]
  cb2v = cb2[...]
  l11 = jnp.maximum(
      lax.dot_general(emb1, cw1[...], dn, preferred_element_type=jnp.float32)
      + cb1v, 0.0)
  l12 = jnp.maximum(
      lax.dot_general(emb2, cw1[...], dn, preferred_element_type=jnp.float32)
      + cb1v, 0.0)
  a1 = lax.dot_general(l11, cw2[...], dn,
                       preferred_element_type=jnp.float32) + cb2v
  a2 = lax.dot_general(l12, cw2[...], dn,
                       preferred_element_type=jnp.float32) + cb2v
  m = jnp.maximum(a1, a2)
  x1 = jnp.exp(a1 - m)
  x2 = jnp.exp(a2 - m)
  center_o[...] = (x1 * emb1 + x2 * emb2) / (x1 + x2)

  o1 = po[:, :D]
  o2 = po[:, D:]
  ob1v = ob1[...]
  oa1 = jnp.maximum(
      lax.dot_general(o1, ow1[...], dn,
                      preferred_element_type=jnp.float32) + ob1v, 0.0)
  oa2 = jnp.maximum(
      lax.dot_general(o2, ow1[...], dn,
                      preferred_element_type=jnp.float32) + ob1v, 0.0)
  omean = 0.5 * (oa1 + oa2)
  gate = jax.nn.sigmoid(
      lax.dot_general(omean, ow2[...], dn,
                      preferred_element_type=jnp.float32) + ob2[...])
  offset_o[...] = jnp.minimum(o1, o2) * gate


def _tc_intersect(eo1, eo2, ro, po, cw1, cb1, cw2, cb2, ow1, ob1, ow2, ob2):
  rows2 = pl.BlockSpec((BK, 2 * D), lambda i: (i, 0))
  rows = pl.BlockSpec((BK, D), lambda i: (i, 0))
  wmat = pl.BlockSpec((D, D), lambda i: (0, 0))
  wvec = pl.BlockSpec((1, D), lambda i: (0, 0))
  return pl.pallas_call(
      _tc_body,
      grid=(B // BK,),
      in_specs=[rows2, rows2, rows2, rows2,
                wmat, wvec, wmat, wvec, wmat, wvec, wmat, wvec],
      out_specs=[rows, rows],
      out_shape=[jax.ShapeDtypeStruct((B, D), jnp.float32)] * 2,
  )(eo1, eo2, ro, po, cw1, cb1, cw2, cb2, ow1, ob1, ow2, ob2)


def kernel(anchor1, rel1, anchor2, rel2, entity_embedding, relation_embedding,
           offset_embedding, c_w1, c_b1, c_w2, c_b2, o_w1, o_b1, o_w2, o_b2):
  a1 = anchor1.astype(jnp.int32)
  a2 = anchor2.astype(jnp.int32)
  r1 = rel1.astype(jnp.int32)
  r2 = rel2.astype(jnp.int32)
  relp = jnp.pad(relation_embedding, ((0, 0), (0, D)))
  offp = jnp.pad(offset_embedding, ((0, 0), (0, D)))
  tailp = jnp.pad(entity_embedding[ETAIL:], ((0, 0), (0, D)))
  ent3 = jnp.reshape(entity_embedding.T, (8, 8, NE))
  ro, po, eo1, eo2 = _sc_gather(a1, r1, a2, r2, ent3, relp, offp, tailp)
  center, offset = _tc_intersect(
      eo1, eo2, ro, po,
      c_w1, c_b1.reshape(1, D), c_w2, c_b2.reshape(1, D),
      o_w1, o_b1.reshape(1, D), o_w2, o_b2.reshape(1, D))
  return (center, offset)


# speedup vs baseline: 1.3094x; 1.3094x over previous
import functools

import jax
import jax.numpy as jnp
from jax import lax
from jax.experimental import pallas as pl
from jax.experimental.pallas import tpu as pltpu
from jax.experimental.pallas import tpu_sc as plsc

B = 16384
D = 64
NC = 2
NS = 16
NW = NC * NS
BPW = B // NW
C = 64
NCHUNK = BPW // C
NV = D // 16


def _sc_relk(r1, r2, relp, offp):
  mesh = plsc.VectorSubcoreMesh(core_axis_name="c", subcore_axis_name="s")
  out_t = jax.ShapeDtypeStruct((B, 2 * D), jnp.float32)

  @functools.partial(
      pl.kernel,
      out_type=[out_t, out_t],
      mesh=mesh,
      compiler_params=pltpu.CompilerParams(needs_layout_passes=False),
      scratch_types=[
          pltpu.VMEM((BPW,), jnp.int32),
          pltpu.VMEM((BPW,), jnp.int32),
          pltpu.VMEM((2, C, 2 * D), jnp.float32),
          pltpu.VMEM((2, C, 2 * D), jnp.float32),
          pltpu.VMEM((2, C, 2 * D), jnp.float32),
          pltpu.VMEM((2, C, 2 * D), jnp.float32),
          pltpu.VMEM((2, C, 2 * D), jnp.float32),
          pltpu.VMEM((2, C, 2 * D), jnp.float32),
          pltpu.SemaphoreType.DMA,
          pltpu.SemaphoreType.DMA,
      ],
  )
  def k(r1_h, r2_h, rel_h, off_h, ro_o, po_o,
        ir1, ir2, rb1, rb2, ob1, ob2, re, po, sem, sem2):
    wid = lax.axis_index("s") * NC + lax.axis_index("c")
    base_w = wid * BPW
    pltpu.sync_copy(r1_h.at[pl.ds(base_w, BPW)], ir1)
    pltpu.sync_copy(r2_h.at[pl.ds(base_w, BPW)], ir2)

    def issue(ci):
      o = ci * C
      p = ci % 2
      pltpu.async_copy(rel_h.at[ir1.at[pl.ds(o, C)]], rb1.at[p], sem)
      pltpu.async_copy(rel_h.at[ir2.at[pl.ds(o, C)]], rb2.at[p], sem)
      pltpu.async_copy(off_h.at[ir1.at[pl.ds(o, C)]], ob1.at[p], sem)
      pltpu.async_copy(off_h.at[ir2.at[pl.ds(o, C)]], ob2.at[p], sem)

    issue(0)
    for ci in range(NCHUNK):
      p = ci % 2
      if ci + 1 < NCHUNK:
        issue(ci + 1)
      for _ in range(4):
        pltpu.make_async_copy(rel_h.at[ir1.at[pl.ds(0, C)]], rb1.at[0],
                              sem).wait()
      if ci >= 2:
        pltpu.make_async_copy(re.at[0], ro_o.at[pl.ds(0, C)], sem2).wait()
        pltpu.make_async_copy(re.at[0], ro_o.at[pl.ds(0, C)], sem2).wait()

      def row_body(rr, _):
        for j in range(NV):
          sl = pl.ds(j * 16, 16)
          sr = pl.ds(D + j * 16, 16)
          re[p, rr, sl] = rb1[p, rr, sl]
          re[p, rr, sr] = rb2[p, rr, sl]
          po[p, rr, sl] = ob1[p, rr, sl]
          po[p, rr, sr] = ob2[p, rr, sl]
        return 0

      lax.fori_loop(0, C, row_body, 0, unroll=2)
      base = base_w + ci * C
      pltpu.async_copy(re.at[p], ro_o.at[pl.ds(base, C)], sem2)
      pltpu.async_copy(po.at[p], po_o.at[pl.ds(base, C)], sem2)
    for _ in range(4):
      pltpu.make_async_copy(re.at[0], ro_o.at[pl.ds(0, C)], sem2).wait()

  return k(r1, r2, relp, offp)


def _sc_entk(a1, a2, ent):
  mesh = plsc.VectorSubcoreMesh(core_axis_name="c", subcore_axis_name="s")
  out_t = jax.ShapeDtypeStruct((B, 2 * D), jnp.float32)

  @functools.partial(
      pl.kernel,
      out_type=[out_t],
      mesh=mesh,
      compiler_params=pltpu.CompilerParams(needs_layout_passes=False),
      scratch_types=[
          pltpu.VMEM((BPW,), jnp.int32),
          pltpu.VMEM((BPW,), jnp.int32),
          pltpu.VMEM((2, C, D), jnp.float32),
          pltpu.VMEM((2, C, D), jnp.float32),
          pltpu.VMEM((2, C, 2 * D), jnp.float32),
          pltpu.SemaphoreType.DMA,
          pltpu.SemaphoreType.DMA,
      ],
  )
  def k(a1_h, a2_h, ent_h, eo_o, ia1, ia2, eb1, eb2, pe, sem, sem2):
    wid = lax.axis_index("s") * NC + lax.axis_index("c")
    base_w = wid * BPW
    pltpu.sync_copy(a1_h.at[pl.ds(base_w, BPW)], ia1)
    pltpu.sync_copy(a2_h.at[pl.ds(base_w, BPW)], ia2)

    def issue(ci):
      o = ci * C
      p = ci % 2

      def grp(g, _):
        v1 = ia1[pl.ds(o + g * 16, 16)]
        v2 = ia2[pl.ds(o + g * 16, 16)]
        for l in range(16):
          q = g * 16 + l
          pltpu.async_copy(ent_h.at[pl.ds(v1[l], 1)],
                           eb1.at[p, pl.ds(q, 1)], sem)
          pltpu.async_copy(ent_h.at[pl.ds(v2[l], 1)],
                           eb2.at[p, pl.ds(q, 1)], sem)
        return 0

      lax.fori_loop(0, C // 16, grp, 0)

    issue(0)
    for ci in range(NCHUNK):
      p = ci % 2
      if ci + 1 < NCHUNK:
        issue(ci + 1)
      for _ in range(2 * C):
        pltpu.make_async_copy(ent_h.at[pl.ds(0, 1)], eb1.at[0, pl.ds(0, 1)],
                              sem).wait()
      if ci >= 2:
        pltpu.make_async_copy(pe.at[0], eo_o.at[pl.ds(0, C)], sem2).wait()

      def row_body(rr, _):
        for j in range(NV):
          sl = pl.ds(j * 16, 16)
          sr = pl.ds(D + j * 16, 16)
          pe[p, rr, sl] = eb1[p, rr, sl]
          pe[p, rr, sr] = eb2[p, rr, sl]
        return 0

      lax.fori_loop(0, C, row_body, 0, unroll=2)
      pltpu.async_copy(pe.at[p], eo_o.at[pl.ds(base_w + ci * C, C)], sem2)
    for _ in range(2):
      pltpu.make_async_copy(pe.at[0], eo_o.at[pl.ds(0, C)], sem2).wait()

  return k(a1, a2, ent)


BK = 2048


def _tc_body(eo, ro, po, cw1, cb1, cw2, cb2, ow1, ob1, ow2, ob2,
             center_o, offset_o):
  dn = (((1,), (1,)), ((), ()))
  emb1 = eo[:, :D] + ro[:, :D]
  emb2 = eo[:, D:] + ro[:, D:]
  cb1v = cb1[...]
  cb2v = cb2[...]
  l11 = jnp.maximum(
      lax.dot_general(emb1, cw1[...], dn, preferred_element_type=jnp.float32)
      + cb1v, 0.0)
  l12 = jnp.maximum(
      lax.dot_general(emb2, cw1[...], dn, preferred_element_type=jnp.float32)
      + cb1v, 0.0)
  a1 = lax.dot_general(l11, cw2[...], dn,
                       preferred_element_type=jnp.float32) + cb2v
  a2 = lax.dot_general(l12, cw2[...], dn,
                       preferred_element_type=jnp.float32) + cb2v
  m = jnp.maximum(a1, a2)
  x1 = jnp.exp(a1 - m)
  x2 = jnp.exp(a2 - m)
  center_o[...] = (x1 * emb1 + x2 * emb2) / (x1 + x2)

  o1 = po[:, :D]
  o2 = po[:, D:]
  ob1v = ob1[...]
  oa1 = jnp.maximum(
      lax.dot_general(o1, ow1[...], dn,
                      preferred_element_type=jnp.float32) + ob1v, 0.0)
  oa2 = jnp.maximum(
      lax.dot_general(o2, ow1[...], dn,
                      preferred_element_type=jnp.float32) + ob1v, 0.0)
  omean = 0.5 * (oa1 + oa2)
  gate = jax.nn.sigmoid(
      lax.dot_general(omean, ow2[...], dn,
                      preferred_element_type=jnp.float32) + ob2[...])
  offset_o[...] = jnp.minimum(o1, o2) * gate


def _tc_intersect(eo, ro, po, cw1, cb1, cw2, cb2, ow1, ob1, ow2, ob2):
  rows2 = pl.BlockSpec((BK, 2 * D), lambda i: (i, 0))
  rows = pl.BlockSpec((BK, D), lambda i: (i, 0))
  wmat = pl.BlockSpec((D, D), lambda i: (0, 0))
  wvec = pl.BlockSpec((1, D), lambda i: (0, 0))
  return pl.pallas_call(
      _tc_body,
      grid=(B // BK,),
      in_specs=[rows2, rows2, rows2,
                wmat, wvec, wmat, wvec, wmat, wvec, wmat, wvec],
      out_specs=[rows, rows],
      out_shape=[jax.ShapeDtypeStruct((B, D), jnp.float32)] * 2,
  )(eo, ro, po, cw1, cb1, cw2, cb2, ow1, ob1, ow2, ob2)


def kernel(anchor1, rel1, anchor2, rel2, entity_embedding, relation_embedding,
           offset_embedding, c_w1, c_b1, c_w2, c_b2, o_w1, o_b1, o_w2, o_b2):
  a1 = anchor1.astype(jnp.int32)
  a2 = anchor2.astype(jnp.int32)
  r1 = rel1.astype(jnp.int32)
  r2 = rel2.astype(jnp.int32)
  relp = jnp.pad(relation_embedding, ((0, 0), (0, D)))
  offp = jnp.pad(offset_embedding, ((0, 0), (0, D)))
  ro, po = _sc_relk(r1, r2, relp, offp)
  eo, = _sc_entk(a1, a2, entity_embedding)
  center, offset = _tc_intersect(
      eo, ro, po,
      c_w1, c_b1.reshape(1, D), c_w2, c_b2.reshape(1, D),
      o_w1, o_b1.reshape(1, D), o_w2, o_b2.reshape(1, D))
  return (center, offset)


# speedup vs baseline: 1.6304x; 1.2451x over previous
import functools

import jax
import jax.numpy as jnp
from jax import lax
from jax.experimental import pallas as pl
from jax.experimental.pallas import tpu as pltpu
from jax.experimental.pallas import tpu_sc as plsc

B = 16384
D = 64
NC = 2
NS = 16
NW = NC * NS
BPW = B // NW
C = 64
NCHUNK = BPW // C
NV = D // 16


def _sc_relk(r1, r2, relp, offp):
  mesh = plsc.VectorSubcoreMesh(core_axis_name="c", subcore_axis_name="s")
  out_t = jax.ShapeDtypeStruct((B, 2 * D), jnp.float32)

  @functools.partial(
      pl.kernel,
      out_type=[out_t, out_t],
      mesh=mesh,
      compiler_params=pltpu.CompilerParams(needs_layout_passes=False),
      scratch_types=[
          pltpu.VMEM((BPW,), jnp.int32),
          pltpu.VMEM((BPW,), jnp.int32),
          pltpu.VMEM((2, C, 2 * D), jnp.float32),
          pltpu.VMEM((2, C, 2 * D), jnp.float32),
          pltpu.VMEM((2, C, 2 * D), jnp.float32),
          pltpu.VMEM((2, C, 2 * D), jnp.float32),
          pltpu.VMEM((2, C, 2 * D), jnp.float32),
          pltpu.VMEM((2, C, 2 * D), jnp.float32),
          pltpu.SemaphoreType.DMA,
          pltpu.SemaphoreType.DMA,
      ],
  )
  def k(r1_h, r2_h, rel_h, off_h, ro_o, po_o,
        ir1, ir2, rb1, rb2, ob1, ob2, re, po, sem, sem2):
    wid = lax.axis_index("s") * NC + lax.axis_index("c")
    base_w = wid * BPW
    pltpu.sync_copy(r1_h.at[pl.ds(base_w, BPW)], ir1)
    pltpu.sync_copy(r2_h.at[pl.ds(base_w, BPW)], ir2)

    def issue(ci):
      o = ci * C
      p = ci % 2
      pltpu.async_copy(rel_h.at[ir1.at[pl.ds(o, C)]], rb1.at[p], sem)
      pltpu.async_copy(rel_h.at[ir2.at[pl.ds(o, C)]], rb2.at[p], sem)
      pltpu.async_copy(off_h.at[ir1.at[pl.ds(o, C)]], ob1.at[p], sem)
      pltpu.async_copy(off_h.at[ir2.at[pl.ds(o, C)]], ob2.at[p], sem)

    issue(0)
    for ci in range(NCHUNK):
      p = ci % 2
      if ci + 1 < NCHUNK:
        issue(ci + 1)
      for _ in range(4):
        pltpu.make_async_copy(rel_h.at[ir1.at[pl.ds(0, C)]], rb1.at[0],
                              sem).wait()
      if ci >= 2:
        pltpu.make_async_copy(re.at[0], ro_o.at[pl.ds(0, C)], sem2).wait()
        pltpu.make_async_copy(re.at[0], ro_o.at[pl.ds(0, C)], sem2).wait()

      def row_body(rr, _):
        for j in range(NV):
          sl = pl.ds(j * 16, 16)
          sr = pl.ds(D + j * 16, 16)
          re[p, rr, sl] = rb1[p, rr, sl]
          re[p, rr, sr] = rb2[p, rr, sl]
          po[p, rr, sl] = ob1[p, rr, sl]
          po[p, rr, sr] = ob2[p, rr, sl]
        return 0

      lax.fori_loop(0, C, row_body, 0, unroll=2)
      base = base_w + ci * C
      pltpu.async_copy(re.at[p], ro_o.at[pl.ds(base, C)], sem2)
      pltpu.async_copy(po.at[p], po_o.at[pl.ds(base, C)], sem2)
    for _ in range(4):
      pltpu.make_async_copy(re.at[0], ro_o.at[pl.ds(0, C)], sem2).wait()

  return k(r1, r2, relp, offp)


def _sc_entk(a1, a2, ent):
  mesh = plsc.VectorSubcoreMesh(core_axis_name="c", subcore_axis_name="s")
  out_t = jax.ShapeDtypeStruct((B, 2 * D), jnp.float32)

  @functools.partial(
      pl.kernel,
      out_type=[out_t],
      mesh=mesh,
      compiler_params=pltpu.CompilerParams(needs_layout_passes=False),
      scratch_types=[
          pltpu.VMEM((BPW,), jnp.int32),
          pltpu.VMEM((BPW,), jnp.int32),
          pltpu.VMEM((2, C, D), jnp.float32),
          pltpu.VMEM((2, C, D), jnp.float32),
          pltpu.VMEM((2, C, 2 * D), jnp.float32),
          pltpu.SemaphoreType.DMA,
          pltpu.SemaphoreType.DMA,
      ],
  )
  def k(a1_h, a2_h, ent_h, eo_o, ia1, ia2, eb1, eb2, pe, sem, sem2):
    wid = lax.axis_index("s") * NC + lax.axis_index("c")
    base_w = wid * BPW
    pltpu.sync_copy(a1_h.at[pl.ds(base_w, BPW)], ia1)
    pltpu.sync_copy(a2_h.at[pl.ds(base_w, BPW)], ia2)

    def issue(ci):
      o = ci * C
      p = ci % 2

      def grp(g, _):
        v1 = ia1[pl.ds(o + g * 16, 16)]
        v2 = ia2[pl.ds(o + g * 16, 16)]
        for l in range(16):
          q = g * 16 + l
          pltpu.async_copy(ent_h.at[pl.ds(v1[l], 1)],
                           eb1.at[p, pl.ds(q, 1)], sem)
          pltpu.async_copy(ent_h.at[pl.ds(v2[l], 1)],
                           eb2.at[p, pl.ds(q, 1)], sem)
        return 0

      lax.fori_loop(0, C // 16, grp, 0)

    issue(0)
    for ci in range(NCHUNK):
      p = ci % 2
      if ci + 1 < NCHUNK:
        issue(ci + 1)
      for _ in range(2 * C):
        pltpu.make_async_copy(ent_h.at[pl.ds(0, 1)], eb1.at[0, pl.ds(0, 1)],
                              sem).wait()
      if ci >= 2:
        pltpu.make_async_copy(pe.at[0], eo_o.at[pl.ds(0, C)], sem2).wait()

      def row_body(rr, _):
        for j in range(NV):
          sl = pl.ds(j * 16, 16)
          sr = pl.ds(D + j * 16, 16)
          pe[p, rr, sl] = eb1[p, rr, sl]
          pe[p, rr, sr] = eb2[p, rr, sl]
        return 0

      lax.fori_loop(0, C, row_body, 0, unroll=2)
      pltpu.async_copy(pe.at[p], eo_o.at[pl.ds(base_w + ci * C, C)], sem2)
    for _ in range(2):
      pltpu.make_async_copy(pe.at[0], eo_o.at[pl.ds(0, C)], sem2).wait()

  return k(a1, a2, ent)


NE = 1000000
TBK = 16384


def _tc_transpose(entt):
  def body(x_ref, o_ref):
    o_ref[...] = x_ref[...].T

  return pl.pallas_call(
      body,
      grid=(pl.cdiv(NE, TBK),),
      in_specs=[pl.BlockSpec((D, TBK), lambda i: (0, i))],
      out_specs=pl.BlockSpec((TBK, D), lambda i: (i, 0)),
      out_shape=jax.ShapeDtypeStruct((NE, D), jnp.float32),
  )(entt)


BK = 2048


def _tc_body(eo, ro, po, cw1, cb1, cw2, cb2, ow1, ob1, ow2, ob2,
             center_o, offset_o):
  dn = (((1,), (1,)), ((), ()))
  emb1 = eo[:, :D] + ro[:, :D]
  emb2 = eo[:, D:] + ro[:, D:]
  cb1v = cb1[...]
  cb2v = cb2[...]
  l11 = jnp.maximum(
      lax.dot_general(emb1, cw1[...], dn, preferred_element_type=jnp.float32)
      + cb1v, 0.0)
  l12 = jnp.maximum(
      lax.dot_general(emb2, cw1[...], dn, preferred_element_type=jnp.float32)
      + cb1v, 0.0)
  a1 = lax.dot_general(l11, cw2[...], dn,
                       preferred_element_type=jnp.float32) + cb2v
  a2 = lax.dot_general(l12, cw2[...], dn,
                       preferred_element_type=jnp.float32) + cb2v
  m = jnp.maximum(a1, a2)
  x1 = jnp.exp(a1 - m)
  x2 = jnp.exp(a2 - m)
  center_o[...] = (x1 * emb1 + x2 * emb2) / (x1 + x2)

  o1 = po[:, :D]
  o2 = po[:, D:]
  ob1v = ob1[...]
  oa1 = jnp.maximum(
      lax.dot_general(o1, ow1[...], dn,
                      preferred_element_type=jnp.float32) + ob1v, 0.0)
  oa2 = jnp.maximum(
      lax.dot_general(o2, ow1[...], dn,
                      preferred_element_type=jnp.float32) + ob1v, 0.0)
  omean = 0.5 * (oa1 + oa2)
  gate = jax.nn.sigmoid(
      lax.dot_general(omean, ow2[...], dn,
                      preferred_element_type=jnp.float32) + ob2[...])
  offset_o[...] = jnp.minimum(o1, o2) * gate


def _tc_intersect(eo, ro, po, cw1, cb1, cw2, cb2, ow1, ob1, ow2, ob2):
  rows2 = pl.BlockSpec((BK, 2 * D), lambda i: (i, 0))
  rows = pl.BlockSpec((BK, D), lambda i: (i, 0))
  wmat = pl.BlockSpec((D, D), lambda i: (0, 0))
  wvec = pl.BlockSpec((1, D), lambda i: (0, 0))
  return pl.pallas_call(
      _tc_body,
      grid=(B // BK,),
      in_specs=[rows2, rows2, rows2,
                wmat, wvec, wmat, wvec, wmat, wvec, wmat, wvec],
      out_specs=[rows, rows],
      out_shape=[jax.ShapeDtypeStruct((B, D), jnp.float32)] * 2,
  )(eo, ro, po, cw1, cb1, cw2, cb2, ow1, ob1, ow2, ob2)


def kernel(anchor1, rel1, anchor2, rel2, entity_embedding, relation_embedding,
           offset_embedding, c_w1, c_b1, c_w2, c_b2, o_w1, o_b1, o_w2, o_b2):
  a1 = anchor1.astype(jnp.int32)
  a2 = anchor2.astype(jnp.int32)
  r1 = rel1.astype(jnp.int32)
  r2 = rel2.astype(jnp.int32)
  relp = jnp.pad(relation_embedding, ((0, 0), (0, D)))
  offp = jnp.pad(offset_embedding, ((0, 0), (0, D)))
  ro, po = _sc_relk(r1, r2, relp, offp)
  ent_rm = _tc_transpose(entity_embedding.T)
  eo, = _sc_entk(a1, a2, ent_rm)
  center, offset = _tc_intersect(
      eo, ro, po,
      c_w1, c_b1.reshape(1, D), c_w2, c_b2.reshape(1, D),
      o_w1, o_b1.reshape(1, D), o_w2, o_b2.reshape(1, D))
  return (center, offset)


# speedup vs baseline: 1.6629x; 1.0200x over previous
import functools

import jax
import jax.numpy as jnp
from jax import lax
from jax.experimental import pallas as pl
from jax.experimental.pallas import tpu as pltpu
from jax.experimental.pallas import tpu_sc as plsc

B = 16384
D = 64
NC = 2
NS = 16
NW = NC * NS
BPW = B // NW
C = 64
NCHUNK = BPW // C
NV = D // 16


def _sc_relk(r1, r2, relp, offp):
  mesh = plsc.VectorSubcoreMesh(core_axis_name="c", subcore_axis_name="s")
  out_t = jax.ShapeDtypeStruct((B, 2 * D), jnp.float32)

  @functools.partial(
      pl.kernel,
      out_type=[out_t, out_t],
      mesh=mesh,
      compiler_params=pltpu.CompilerParams(needs_layout_passes=False),
      scratch_types=[
          pltpu.VMEM((BPW,), jnp.int32),
          pltpu.VMEM((BPW,), jnp.int32),
          pltpu.VMEM((2, C, 2 * D), jnp.float32),
          pltpu.VMEM((2, C, 2 * D), jnp.float32),
          pltpu.VMEM((2, C, 2 * D), jnp.float32),
          pltpu.VMEM((2, C, 2 * D), jnp.float32),
          pltpu.VMEM((2, C, 2 * D), jnp.float32),
          pltpu.VMEM((2, C, 2 * D), jnp.float32),
          pltpu.SemaphoreType.DMA,
          pltpu.SemaphoreType.DMA,
      ],
  )
  def k(r1_h, r2_h, rel_h, off_h, ro_o, po_o,
        ir1, ir2, rb1, rb2, ob1, ob2, re, po, sem, sem2):
    wid = lax.axis_index("s") * NC + lax.axis_index("c")
    base_w = wid * BPW
    pltpu.sync_copy(r1_h.at[pl.ds(base_w, BPW)], ir1)
    pltpu.sync_copy(r2_h.at[pl.ds(base_w, BPW)], ir2)

    def issue(ci):
      o = ci * C
      p = ci % 2
      pltpu.async_copy(rel_h.at[ir1.at[pl.ds(o, C)]], rb1.at[p], sem)
      pltpu.async_copy(rel_h.at[ir2.at[pl.ds(o, C)]], rb2.at[p], sem)
      pltpu.async_copy(off_h.at[ir1.at[pl.ds(o, C)]], ob1.at[p], sem)
      pltpu.async_copy(off_h.at[ir2.at[pl.ds(o, C)]], ob2.at[p], sem)

    issue(0)
    for ci in range(NCHUNK):
      p = ci % 2
      if ci + 1 < NCHUNK:
        issue(ci + 1)
      for _ in range(4):
        pltpu.make_async_copy(rel_h.at[ir1.at[pl.ds(0, C)]], rb1.at[0],
                              sem).wait()
      if ci >= 2:
        pltpu.make_async_copy(re.at[0], ro_o.at[pl.ds(0, C)], sem2).wait()
        pltpu.make_async_copy(re.at[0], ro_o.at[pl.ds(0, C)], sem2).wait()

      def row_body(rr, _):
        for j in range(NV):
          sl = pl.ds(j * 16, 16)
          sr = pl.ds(D + j * 16, 16)
          re[p, rr, sl] = rb1[p, rr, sl]
          re[p, rr, sr] = rb2[p, rr, sl]
          po[p, rr, sl] = ob1[p, rr, sl]
          po[p, rr, sr] = ob2[p, rr, sl]
        return 0

      lax.fori_loop(0, C, row_body, 0, unroll=2)
      base = base_w + ci * C
      pltpu.async_copy(re.at[p], ro_o.at[pl.ds(base, C)], sem2)
      pltpu.async_copy(po.at[p], po_o.at[pl.ds(base, C)], sem2)
    for _ in range(4):
      pltpu.make_async_copy(re.at[0], ro_o.at[pl.ds(0, C)], sem2).wait()

  return k(r1, r2, relp, offp)


def _sc_entk(a1, a2, ent):
  mesh = plsc.VectorSubcoreMesh(core_axis_name="c", subcore_axis_name="s")
  out_t = jax.ShapeDtypeStruct((B, 2 * D), jnp.float32)

  @functools.partial(
      pl.kernel,
      out_type=[out_t],
      mesh=mesh,
      compiler_params=pltpu.CompilerParams(needs_layout_passes=False),
      scratch_types=[
          pltpu.VMEM((BPW,), jnp.int32),
          pltpu.VMEM((BPW,), jnp.int32),
          pltpu.VMEM((2, C, D), jnp.float32),
          pltpu.VMEM((2, C, D), jnp.float32),
          pltpu.VMEM((2, C, 2 * D), jnp.float32),
          pltpu.SemaphoreType.DMA,
          pltpu.SemaphoreType.DMA,
      ],
  )
  def k(a1_h, a2_h, ent_h, eo_o, ia1, ia2, eb1, eb2, pe, sem, sem2):
    wid = lax.axis_index("s") * NC + lax.axis_index("c")
    base_w = wid * BPW
    pltpu.sync_copy(a1_h.at[pl.ds(base_w, BPW)], ia1)
    pltpu.sync_copy(a2_h.at[pl.ds(base_w, BPW)], ia2)

    def issue(ci):
      o = ci * C
      p = ci % 2

      def grp(g, _):
        v1 = ia1[pl.ds(o + g * 16, 16)]
        v2 = ia2[pl.ds(o + g * 16, 16)]
        for l in range(16):
          q = g * 16 + l
          pltpu.async_copy(ent_h.at[pl.ds(v1[l], 1)],
                           eb1.at[p, pl.ds(q, 1)], sem)
          pltpu.async_copy(ent_h.at[pl.ds(v2[l], 1)],
                           eb2.at[p, pl.ds(q, 1)], sem)
        return 0

      lax.fori_loop(0, C // 16, grp, 0)

    issue(0)
    for ci in range(NCHUNK):
      p = ci % 2
      if ci + 1 < NCHUNK:
        issue(ci + 1)
      for _ in range(2 * C):
        pltpu.make_async_copy(ent_h.at[pl.ds(0, 1)], eb1.at[0, pl.ds(0, 1)],
                              sem).wait()
      if ci >= 2:
        pltpu.make_async_copy(pe.at[0], eo_o.at[pl.ds(0, C)], sem2).wait()

      def row_body(rr, _):
        for j in range(NV):
          sl = pl.ds(j * 16, 16)
          sr = pl.ds(D + j * 16, 16)
          pe[p, rr, sl] = eb1[p, rr, sl]
          pe[p, rr, sr] = eb2[p, rr, sl]
        return 0

      lax.fori_loop(0, C, row_body, 0, unroll=2)
      pltpu.async_copy(pe.at[p], eo_o.at[pl.ds(base_w + ci * C, C)], sem2)
    for _ in range(2):
      pltpu.make_async_copy(pe.at[0], eo_o.at[pl.ds(0, C)], sem2).wait()

  return k(a1, a2, ent)


NE = 1000000
TBK = 32768


def _tc_transpose(entt):
  def body(x_ref, o_ref):
    o_ref[...] = x_ref[...].T

  return pl.pallas_call(
      body,
      grid=(pl.cdiv(NE, TBK),),
      in_specs=[pl.BlockSpec((D, TBK), lambda i: (0, i))],
      out_specs=pl.BlockSpec((TBK, D), lambda i: (i, 0)),
      out_shape=jax.ShapeDtypeStruct((NE, D), jnp.float32),
  )(entt)


BK = 2048


def _tc_body(eo, ro, po, cw1, cb1, cw2, cb2, ow1, ob1, ow2, ob2,
             center_o, offset_o):
  dn = (((1,), (1,)), ((), ()))
  emb1 = eo[:, :D] + ro[:, :D]
  emb2 = eo[:, D:] + ro[:, D:]
  cb1v = cb1[...]
  cb2v = cb2[...]
  l11 = jnp.maximum(
      lax.dot_general(emb1, cw1[...], dn, preferred_element_type=jnp.float32)
      + cb1v, 0.0)
  l12 = jnp.maximum(
      lax.dot_general(emb2, cw1[...], dn, preferred_element_type=jnp.float32)
      + cb1v, 0.0)
  a1 = lax.dot_general(l11, cw2[...], dn,
                       preferred_element_type=jnp.float32) + cb2v
  a2 = lax.dot_general(l12, cw2[...], dn,
                       preferred_element_type=jnp.float32) + cb2v
  m = jnp.maximum(a1, a2)
  x1 = jnp.exp(a1 - m)
  x2 = jnp.exp(a2 - m)
  center_o[...] = (x1 * emb1 + x2 * emb2) / (x1 + x2)

  o1 = po[:, :D]
  o2 = po[:, D:]
  ob1v = ob1[...]
  oa1 = jnp.maximum(
      lax.dot_general(o1, ow1[...], dn,
                      preferred_element_type=jnp.float32) + ob1v, 0.0)
  oa2 = jnp.maximum(
      lax.dot_general(o2, ow1[...], dn,
                      preferred_element_type=jnp.float32) + ob1v, 0.0)
  omean = 0.5 * (oa1 + oa2)
  gate = jax.nn.sigmoid(
      lax.dot_general(omean, ow2[...], dn,
                      preferred_element_type=jnp.float32) + ob2[...])
  offset_o[...] = jnp.minimum(o1, o2) * gate


def _tc_intersect(eo, ro, po, cw1, cb1, cw2, cb2, ow1, ob1, ow2, ob2):
  rows2 = pl.BlockSpec((BK, 2 * D), lambda i: (i, 0))
  rows = pl.BlockSpec((BK, D), lambda i: (i, 0))
  wmat = pl.BlockSpec((D, D), lambda i: (0, 0))
  wvec = pl.BlockSpec((1, D), lambda i: (0, 0))
  return pl.pallas_call(
      _tc_body,
      grid=(B // BK,),
      in_specs=[rows2, rows2, rows2,
                wmat, wvec, wmat, wvec, wmat, wvec, wmat, wvec],
      out_specs=[rows, rows],
      out_shape=[jax.ShapeDtypeStruct((B, D), jnp.float32)] * 2,
  )(eo, ro, po, cw1, cb1, cw2, cb2, ow1, ob1, ow2, ob2)


def kernel(anchor1, rel1, anchor2, rel2, entity_embedding, relation_embedding,
           offset_embedding, c_w1, c_b1, c_w2, c_b2, o_w1, o_b1, o_w2, o_b2):
  a1 = anchor1.astype(jnp.int32)
  a2 = anchor2.astype(jnp.int32)
  r1 = rel1.astype(jnp.int32)
  r2 = rel2.astype(jnp.int32)
  relp = jnp.pad(relation_embedding, ((0, 0), (0, D)))
  offp = jnp.pad(offset_embedding, ((0, 0), (0, D)))
  ro, po = _sc_relk(r1, r2, relp, offp)
  ent_rm = _tc_transpose(entity_embedding.T)
  eo, = _sc_entk(a1, a2, ent_rm)
  center, offset = _tc_intersect(
      eo, ro, po,
      c_w1, c_b1.reshape(1, D), c_w2, c_b2.reshape(1, D),
      o_w1, o_b1.reshape(1, D), o_w2, o_b2.reshape(1, D))
  return (center, offset)
